# baseline (speedup 1.0000x reference)
import functools

import jax
import jax.numpy as jnp
from jax import lax
from jax.experimental import pallas as pl
from jax.experimental.pallas import tpu as pltpu
from jax.experimental.pallas import tpu_sc as plsc

NUM_CLUSTERS = 8192
EMB_DIM = 256
L = 16
NC = 2
NS = 16
NW = NC * NS
R = NUM_CLUSTERS // NW
NCH = EMB_DIM // L


def _router_body(z_hbm, cent_hbm, val_out, idx_out, z_v, rows_v, val_v, idx_v):
    c = lax.axis_index("c")
    s = lax.axis_index("s")
    wid = c * NS + s
    base = wid * R

    pltpu.sync_copy(z_hbm, z_v)
    pltpu.sync_copy(cent_hbm.at[pl.ds(base, R), :], rows_v)

    zc = [z_v[pl.ds(k * L, L)] for k in range(NCH)]
    zsq_acc = zc[0] * zc[0]
    for k in range(1, NCH):
        zsq_acc = zsq_acc + zc[k] * zc[k]
    zsq_scan = plsc.cumsum(zsq_acc)

    neg_inf = jnp.full((L,), -jnp.inf, dtype=jnp.float32)
    zero_idx = jnp.zeros((L,), dtype=jnp.int32)
    eps = jnp.full((L,), 1e-16, dtype=jnp.float32)

    def row_step(r, carry):
        vbest, vbidx = carry
        v0 = rows_v[r, pl.ds(0, L)]
        dacc = v0 * zc[0]
        sacc = v0 * v0
        for k in range(1, NCH):
            v = rows_v[r, pl.ds(k * L, L)]
            dacc = dacc + v * zc[k]
            sacc = sacc + v * v
        dscan = plsc.cumsum(dacc)
        sscan = plsc.cumsum(sacc)
        t = dscan * jnp.abs(dscan) / jnp.maximum(zsq_scan * sscan, eps)
        m = t > vbest
        ridx = zero_idx + (base + r)
        return jnp.where(m, t, vbest), jnp.where(m, ridx, vbidx)

    vbest, vbidx = lax.fori_loop(0, R, row_step, (neg_inf, zero_idx))
    val_v[...] = vbest
    idx_v[...] = vbidx
    pltpu.sync_copy(val_v, val_out.at[wid])
    pltpu.sync_copy(idx_v, idx_out.at[wid])


_router = pl.kernel(
    _router_body,
    mesh=plsc.VectorSubcoreMesh(core_axis_name="c", subcore_axis_name="s"),
    compiler_params=pltpu.CompilerParams(needs_layout_passes=False),
    out_type=[
        jax.ShapeDtypeStruct((NW, L), jnp.float32),
        jax.ShapeDtypeStruct((NW, L), jnp.int32),
    ],
    scratch_types=[
        pltpu.VMEM((EMB_DIM,), jnp.float32),
        pltpu.VMEM((R, EMB_DIM), jnp.float32),
        pltpu.VMEM((L,), jnp.float32),
        pltpu.VMEM((L,), jnp.int32),
    ],
)


@jax.jit
def kernel(z, centroids):
    vals, idxs = _router(z, centroids)
    t = jnp.argmax(vals[:, L - 1])
    return idxs[t, L - 1]
